# initial kernel scaffold (unmeasured)
import jax
import jax.numpy as jnp
from jax import lax
from jax.experimental import pallas as pl
from jax.experimental.pallas import tpu as pltpu


def _peer():
    return (lax.axis_index("x"), 1 - lax.axis_index("y"), lax.axis_index("z"))


def _exchange_kernel(xb, a2):
    T, D = xb.shape

    def body(x_ref, a_ref, xall_ref, aall_ref, sx_send, sx_recv, sa_send, sa_recv):
        peer = _peer()
        xall_ref[0] = x_ref[...]
        aall_ref[0] = a_ref[...]
        rx = pltpu.make_async_remote_copy(
            src_ref=x_ref,
            dst_ref=xall_ref.at[1],
            send_sem=sx_send,
            recv_sem=sx_recv,
            device_id=peer,
            device_id_type=pl.DeviceIdType.MESH,
        )
        ra = pltpu.make_async_remote_copy(
            src_ref=a_ref,
            dst_ref=aall_ref.at[1],
            send_sem=sa_send,
            recv_sem=sa_recv,
            device_id=peer,
            device_id_type=pl.DeviceIdType.MESH,
        )
        rx.start()
        ra.start()
        rx.wait()
        ra.wait()

    return pl.pallas_call(
        body,
        out_shape=(
            jax.ShapeDtypeStruct((2, T, D), jnp.bfloat16),
            jax.ShapeDtypeStruct((2, T, 1), jnp.int32),
        ),
        in_specs=[
            pl.BlockSpec(memory_space=pltpu.VMEM),
            pl.BlockSpec(memory_space=pltpu.VMEM),
        ],
        out_specs=(
            pl.BlockSpec(memory_space=pltpu.VMEM),
            pl.BlockSpec(memory_space=pltpu.VMEM),
        ),
        scratch_shapes=[
            pltpu.SemaphoreType.DMA,
            pltpu.SemaphoreType.DMA,
            pltpu.SemaphoreType.DMA,
            pltpu.SemaphoreType.DMA,
        ],
        compiler_params=pltpu.CompilerParams(collective_id=0),
    )(xb, a2)


def _moe_kernel(xall, aall, W1b, W2b, MT=512, FT=1024):
    _, T, D = xall.shape
    E_loc, _, F = W1b.shape
    n_m = 2 * T // MT
    n_f = F // FT
    tiles_per_slot = T // MT

    def body(x_ref, a_ref, w1_ref, w2_ref, out_ref, acc_ref):
        e = pl.program_id(1)
        f = pl.program_id(2)
        my_y = lax.axis_index("y")
        ge = my_y * E_loc + e

        mask = a_ref[0] == ge
        xm = jnp.where(mask, x_ref[0], jnp.bfloat16(0))
        h = jnp.dot(xm, w1_ref[0], preferred_element_type=jnp.float32)
        h = jnp.maximum(h, 0.0).astype(jnp.bfloat16)
        part = jnp.dot(h, w2_ref[0], preferred_element_type=jnp.float32)

        first = jnp.logical_and(e == 0, f == 0)
        last = jnp.logical_and(e == E_loc - 1, f == n_f - 1)

        @pl.when(first)
        def _():
            acc_ref[...] = part

        @pl.when(jnp.logical_not(first))
        def _():
            acc_ref[...] += part

        @pl.when(last)
        def _():
            out_ref[0] = acc_ref[...].astype(jnp.bfloat16)

    return pl.pallas_call(
        body,
        grid=(n_m, E_loc, n_f),
        out_shape=jax.ShapeDtypeStruct((2, T, D), jnp.bfloat16),
        in_specs=[
            pl.BlockSpec(
                (1, MT, D), lambda m, e, f: (m // tiles_per_slot, m % tiles_per_slot, 0)
            ),
            pl.BlockSpec(
                (1, MT, 1), lambda m, e, f: (m // tiles_per_slot, m % tiles_per_slot, 0)
            ),
            pl.BlockSpec((1, D, FT), lambda m, e, f: (e, 0, f)),
            pl.BlockSpec((1, FT, D), lambda m, e, f: (e, f, 0)),
        ],
        out_specs=pl.BlockSpec(
            (1, MT, D), lambda m, e, f: (m // tiles_per_slot, m % tiles_per_slot, 0)
        ),
        scratch_shapes=[pltpu.VMEM((MT, D), jnp.float32)],
        compiler_params=pltpu.CompilerParams(
            dimension_semantics=("arbitrary", "arbitrary", "arbitrary"),
        ),
    )(xall, aall, W1b, W2b)


def _combine_kernel(contrib):
    _, T, D = contrib.shape

    def body(c_ref, out_ref, recv_ref, copy_sem, send_sem, recv_sem):
        peer = _peer()
        local = pltpu.make_async_copy(c_ref.at[0], out_ref, copy_sem)
        local.start()
        rdma = pltpu.make_async_remote_copy(
            src_ref=c_ref.at[1],
            dst_ref=recv_ref,
            send_sem=send_sem,
            recv_sem=recv_sem,
            device_id=peer,
            device_id_type=pl.DeviceIdType.MESH,
        )
        rdma.start()
        local.wait()
        rdma.wait()
        out_ref[...] = out_ref[...] + recv_ref[...]

    return pl.pallas_call(
        body,
        out_shape=jax.ShapeDtypeStruct((T, D), jnp.bfloat16),
        in_specs=[pl.BlockSpec(memory_space=pltpu.ANY)],
        out_specs=pl.BlockSpec(memory_space=pltpu.VMEM),
        scratch_shapes=[
            pltpu.VMEM((T, D), jnp.bfloat16),
            pltpu.SemaphoreType.DMA,
            pltpu.SemaphoreType.DMA,
            pltpu.SemaphoreType.DMA,
        ],
        compiler_params=pltpu.CompilerParams(collective_id=1),
    )(contrib)


def kernel(x, assign, W1, W2):
    T, D = x.shape
    xb = x.astype(jnp.bfloat16)
    a2 = assign.reshape(T, 1)
    W1b = W1.astype(jnp.bfloat16)
    W2b = W2.astype(jnp.bfloat16)

    xall, aall = _exchange_kernel(xb, a2)
    contrib = _moe_kernel(xall, aall, W1b, W2b)
    out = _combine_kernel(contrib)
    return out.astype(jnp.float32)


# baseline (device time: 1893368 ns/iter reference)
import jax
import jax.numpy as jnp
from jax import lax
from jax.experimental import pallas as pl
from jax.experimental.pallas import tpu as pltpu


def _peer():
    return (lax.axis_index("x"), 1 - lax.axis_index("y"), lax.axis_index("z"))


def _exchange_kernel(xb, a2):
    T, D = xb.shape

    def body(x_ref, a_ref, xall_ref, aall_ref, cx_sem, ca_sem, sx_send, sx_recv, sa_send, sa_recv):
        peer = _peer()
        cx = pltpu.make_async_copy(x_ref, xall_ref.at[0], cx_sem)
        ca = pltpu.make_async_copy(a_ref, aall_ref.at[0], ca_sem)
        cx.start()
        ca.start()
        rx = pltpu.make_async_remote_copy(
            src_ref=x_ref,
            dst_ref=xall_ref.at[1],
            send_sem=sx_send,
            recv_sem=sx_recv,
            device_id=peer,
            device_id_type=pl.DeviceIdType.MESH,
        )
        ra = pltpu.make_async_remote_copy(
            src_ref=a_ref,
            dst_ref=aall_ref.at[1],
            send_sem=sa_send,
            recv_sem=sa_recv,
            device_id=peer,
            device_id_type=pl.DeviceIdType.MESH,
        )
        rx.start()
        ra.start()
        cx.wait()
        ca.wait()
        rx.wait()
        ra.wait()

    return pl.pallas_call(
        body,
        out_shape=(
            jax.ShapeDtypeStruct((2, T, D), jnp.bfloat16),
            jax.ShapeDtypeStruct((2, T, 1), jnp.int32),
        ),
        in_specs=[
            pl.BlockSpec(memory_space=pltpu.MemorySpace.VMEM),
            pl.BlockSpec(memory_space=pltpu.MemorySpace.VMEM),
        ],
        out_specs=(
            pl.BlockSpec(memory_space=pltpu.MemorySpace.HBM),
            pl.BlockSpec(memory_space=pltpu.MemorySpace.HBM),
        ),
        scratch_shapes=[
            pltpu.SemaphoreType.DMA,
            pltpu.SemaphoreType.DMA,
            pltpu.SemaphoreType.DMA,
            pltpu.SemaphoreType.DMA,
            pltpu.SemaphoreType.DMA,
            pltpu.SemaphoreType.DMA,
        ],
    )(xb, a2)


def _moe_kernel(xall, aall, W1b, W2b, MT=512, FT=1024):
    _, T, D = xall.shape
    E_loc, _, F = W1b.shape
    n_m = 2 * T // MT
    n_f = F // FT
    tiles_per_slot = T // MT

    def body(x_ref, a_ref, w1_ref, w2_ref, out_ref, acc_ref):
        e = pl.program_id(1)
        f = pl.program_id(2)
        my_y = lax.axis_index("y")
        ge = my_y * E_loc + e

        mask = a_ref[0] == ge
        xm = jnp.where(mask, x_ref[0], jnp.bfloat16(0))
        h = jnp.dot(xm, w1_ref[0], preferred_element_type=jnp.float32)
        h = jnp.maximum(h, 0.0).astype(jnp.bfloat16)
        part = jnp.dot(h, w2_ref[0], preferred_element_type=jnp.float32)

        first = jnp.logical_and(e == 0, f == 0)
        last = jnp.logical_and(e == E_loc - 1, f == n_f - 1)

        @pl.when(first)
        def _():
            acc_ref[...] = part

        @pl.when(jnp.logical_not(first))
        def _():
            acc_ref[...] += part

        @pl.when(last)
        def _():
            out_ref[0] = acc_ref[...].astype(jnp.bfloat16)

    return pl.pallas_call(
        body,
        grid=(n_m, E_loc, n_f),
        out_shape=jax.ShapeDtypeStruct((2, T, D), jnp.bfloat16),
        in_specs=[
            pl.BlockSpec(
                (1, MT, D), lambda m, e, f: (m // tiles_per_slot, m % tiles_per_slot, 0)
            ),
            pl.BlockSpec(
                (1, MT, 1), lambda m, e, f: (m // tiles_per_slot, m % tiles_per_slot, 0)
            ),
            pl.BlockSpec((1, D, FT), lambda m, e, f: (e, 0, f)),
            pl.BlockSpec((1, FT, D), lambda m, e, f: (e, f, 0)),
        ],
        out_specs=pl.BlockSpec(
            (1, MT, D), lambda m, e, f: (m // tiles_per_slot, m % tiles_per_slot, 0)
        ),
        scratch_shapes=[pltpu.VMEM((MT, D), jnp.float32)],
        compiler_params=pltpu.CompilerParams(
            dimension_semantics=("arbitrary", "arbitrary", "arbitrary"),
        ),
    )(xall, aall, W1b, W2b)


def _combine_kernel(contrib):
    _, T, D = contrib.shape

    def body(c_ref, out_ref, recv_ref, copy_sem, send_sem, recv_sem):
        peer = _peer()
        local = pltpu.make_async_copy(c_ref.at[0], out_ref, copy_sem)
        local.start()
        rdma = pltpu.make_async_remote_copy(
            src_ref=c_ref.at[1],
            dst_ref=recv_ref,
            send_sem=send_sem,
            recv_sem=recv_sem,
            device_id=peer,
            device_id_type=pl.DeviceIdType.MESH,
        )
        rdma.start()
        local.wait()
        rdma.wait()
        out_ref[...] = out_ref[...] + recv_ref[...]

    return pl.pallas_call(
        body,
        out_shape=jax.ShapeDtypeStruct((T, D), jnp.bfloat16),
        in_specs=[pl.BlockSpec(memory_space=pltpu.MemorySpace.HBM)],
        out_specs=pl.BlockSpec(memory_space=pltpu.MemorySpace.VMEM),
        scratch_shapes=[
            pltpu.VMEM((T, D), jnp.bfloat16),
            pltpu.SemaphoreType.DMA,
            pltpu.SemaphoreType.DMA,
            pltpu.SemaphoreType.DMA,
        ],
        compiler_params=pltpu.CompilerParams(vmem_limit_bytes=50 * 2**20),
    )(contrib)


def kernel(x, assign, W1, W2):
    T, D = x.shape
    xb = x.astype(jnp.bfloat16)
    a2 = assign.reshape(T, 1)
    W1b = W1.astype(jnp.bfloat16)
    W2b = W2.astype(jnp.bfloat16)

    xall, aall = _exchange_kernel(xb, a2)
    contrib = _moe_kernel(xall, aall, W1b, W2b)
    out = _combine_kernel(contrib)
    return out.astype(jnp.float32)
